# baseline (speedup 1.0000x reference)
import jax
import jax.numpy as jnp
from jax import lax
from jax.experimental import pallas as pl
from jax.experimental.pallas import tpu as pltpu

_NCH = 8


def _body(tok_ref, intent_ref, emb_hbm,
          dW_ref, db_ref, sW_ref, sb_ref, ddW_ref, ddb_ref, sdW_ref, sdb_ref,
          intent_out, slots_out, xbuf, sems):
    nj = pl.num_programs(1)
    S = tok_ref.shape[1]
    rows = xbuf.shape[0]
    CS = rows // nj
    C = CS // S
    c = pl.program_id(0)
    j = pl.program_id(1)
    b0 = c * (nj * C)

    @pl.when(j == 0)
    def _issue_all():
        for wi in range(rows):
            tok = tok_ref[b0 + wi // S, wi % S]
            pltpu.make_async_copy(
                emb_hbm.at[pl.ds(tok, 1), :],
                xbuf.at[pl.ds(wi, 1), :],
                sems.at[wi // CS]).start()

    pltpu.make_async_copy(emb_hbm.at[pl.ds(0, CS), :],
                          xbuf.at[pl.ds(0, CS), :], sems.at[j]).wait()

    base = pl.multiple_of(j * CS, 8)
    x = xbuf[pl.ds(base, CS), :]

    we = jnp.maximum(
        jnp.dot(x, sW_ref[...], preferred_element_type=jnp.float32)
        + sb_ref[...], 0.0)

    top = jnp.dot(we, sdW_ref[0:256, :], preferred_element_type=jnp.float32)

    first = b0 + j * C
    n_int = sdW_ref.shape[0] - 256
    iota = lax.broadcasted_iota(jnp.int32, (1, n_int), 1)
    oh = jnp.concatenate(
        [(iota == intent_ref[first + r]).astype(jnp.float32)
         for r in range(C)], axis=0)
    bots = jnp.dot(oh, sdW_ref[256:, :], preferred_element_type=jnp.float32)

    n_slots = top.shape[1]
    top3 = top.reshape(C, S, n_slots)
    slots_out[...] = (top3 + bots[:, None, :] + sdb_ref[...]).reshape(CS, n_slots)

    m = jnp.mean(x.reshape(C, S, x.shape[1]), axis=1)
    se = jnp.maximum(
        jnp.dot(m, dW_ref[...], preferred_element_type=jnp.float32)
        + db_ref[...], 0.0)
    logits = (jnp.dot(se, ddW_ref[...], preferred_element_type=jnp.float32)
              + ddb_ref[...])
    intent_out[...] = logits.reshape(intent_out.shape)


def kernel(token_ids, all_intents, emb_table, doc_enc_W, doc_enc_b,
           slot_enc_W, slot_enc_b, doc_dec_W, doc_dec_b,
           slot_dec_W, slot_dec_b):
    B, S = token_ids.shape
    VOCAB, EMB = emb_table.shape
    ENC = doc_enc_W.shape[1]
    N_INTENTS = doc_dec_W.shape[1]
    N_SLOTS = slot_dec_W.shape[1]
    nj = _NCH
    half = B // 2
    C = half // nj

    tok = token_ids.astype(jnp.int32)
    intents = all_intents.astype(jnp.int32)

    grid = (2, nj)

    def _fixed(c, j, *_):
        return (0, 0)

    def _slots_map(c, j, *_):
        return (c * nj + j, 0)

    def _intent_map(c, j, *_):
        return (c * nj + j, 0, 0)

    grid_spec = pltpu.PrefetchScalarGridSpec(
        num_scalar_prefetch=2,
        grid=grid,
        in_specs=[
            pl.BlockSpec(memory_space=pl.ANY),
            pl.BlockSpec((EMB, ENC), _fixed),
            pl.BlockSpec((1, ENC), _fixed),
            pl.BlockSpec((EMB, ENC), _fixed),
            pl.BlockSpec((1, ENC), _fixed),
            pl.BlockSpec((ENC, N_INTENTS), _fixed),
            pl.BlockSpec((1, N_INTENTS), _fixed),
            pl.BlockSpec((slot_dec_W.shape[0], N_SLOTS), _fixed),
            pl.BlockSpec((1, N_SLOTS), _fixed),
        ],
        out_specs=[
            pl.BlockSpec((C, 1, N_INTENTS), _intent_map),
            pl.BlockSpec((C * S, N_SLOTS), _slots_map),
        ],
        scratch_shapes=[
            pltpu.VMEM((half * S, EMB), jnp.float32),
            pltpu.SemaphoreType.DMA((nj,)),
        ],
    )

    intent_batch, slots_batch = pl.pallas_call(
        _body,
        out_shape=[
            jax.ShapeDtypeStruct((B, 1, N_INTENTS), jnp.float32),
            jax.ShapeDtypeStruct((B * S, N_SLOTS), jnp.float32),
        ],
        grid_spec=grid_spec,
        compiler_params=pltpu.CompilerParams(
            dimension_semantics=("parallel", "arbitrary"),
            disable_bounds_checks=True,
        ),
        name="intent_slot_fused",
    )(tok, intents, emb_table, doc_enc_W, doc_enc_b.reshape(1, ENC),
      slot_enc_W, slot_enc_b.reshape(1, ENC), doc_dec_W,
      doc_dec_b.reshape(1, N_INTENTS), slot_dec_W,
      slot_dec_b.reshape(1, N_SLOTS))
    return intent_batch.reshape(B, N_INTENTS), slots_batch


# speedup vs baseline: 1.0504x; 1.0504x over previous
import jax
import jax.numpy as jnp
from jax import lax
from jax.experimental import pallas as pl
from jax.experimental.pallas import tpu as pltpu

_NCH = 2


def _body(tok_ref, intent_ref, emb_hbm,
          dW_ref, db_ref, sW_ref, sb_ref, ddW_ref, ddb_ref, sdW_ref, sdb_ref,
          intent_out, slots_out, xbuf, sems):
    nj = pl.num_programs(1)
    S = tok_ref.shape[1]
    rows = xbuf.shape[0]
    CS = rows // nj
    C = CS // S
    c = pl.program_id(0)
    j = pl.program_id(1)
    b0 = c * (nj * C)

    @pl.when(j == 0)
    def _issue_all():
        for wi in range(rows):
            tok = tok_ref[b0 + wi // S, wi % S]
            pltpu.make_async_copy(
                emb_hbm.at[pl.ds(tok, 1), :],
                xbuf.at[pl.ds(wi, 1), :],
                sems.at[wi // CS]).start()

    pltpu.make_async_copy(emb_hbm.at[pl.ds(0, CS), :],
                          xbuf.at[pl.ds(0, CS), :], sems.at[j]).wait()

    base = pl.multiple_of(j * CS, 8)
    x = xbuf[pl.ds(base, CS), :]

    we = jnp.maximum(
        jnp.dot(x, sW_ref[...], preferred_element_type=jnp.float32)
        + sb_ref[...], 0.0)

    top = jnp.dot(we, sdW_ref[0:256, :], preferred_element_type=jnp.float32)

    first = b0 + j * C
    n_int = sdW_ref.shape[0] - 256
    iota = lax.broadcasted_iota(jnp.int32, (1, n_int), 1)
    oh = jnp.concatenate(
        [(iota == intent_ref[first + r]).astype(jnp.float32)
         for r in range(C)], axis=0)
    bots = jnp.dot(oh, sdW_ref[256:, :], preferred_element_type=jnp.float32)

    n_slots = top.shape[1]
    top3 = top.reshape(C, S, n_slots)
    slots_out[...] = (top3 + bots[:, None, :] + sdb_ref[...]).reshape(CS, n_slots)

    m = jnp.mean(x.reshape(C, S, x.shape[1]), axis=1)
    se = jnp.maximum(
        jnp.dot(m, dW_ref[...], preferred_element_type=jnp.float32)
        + db_ref[...], 0.0)
    logits = (jnp.dot(se, ddW_ref[...], preferred_element_type=jnp.float32)
              + ddb_ref[...])
    intent_out[...] = logits.reshape(intent_out.shape)


def kernel(token_ids, all_intents, emb_table, doc_enc_W, doc_enc_b,
           slot_enc_W, slot_enc_b, doc_dec_W, doc_dec_b,
           slot_dec_W, slot_dec_b):
    B, S = token_ids.shape
    VOCAB, EMB = emb_table.shape
    ENC = doc_enc_W.shape[1]
    N_INTENTS = doc_dec_W.shape[1]
    N_SLOTS = slot_dec_W.shape[1]
    nj = _NCH
    half = B // 2
    C = half // nj

    tok = token_ids.astype(jnp.int32)
    intents = all_intents.astype(jnp.int32)

    grid = (2, nj)

    def _fixed(c, j, *_):
        return (0, 0)

    def _slots_map(c, j, *_):
        return (c * nj + j, 0)

    def _intent_map(c, j, *_):
        return (c * nj + j, 0, 0)

    grid_spec = pltpu.PrefetchScalarGridSpec(
        num_scalar_prefetch=2,
        grid=grid,
        in_specs=[
            pl.BlockSpec(memory_space=pl.ANY),
            pl.BlockSpec((EMB, ENC), _fixed),
            pl.BlockSpec((1, ENC), _fixed),
            pl.BlockSpec((EMB, ENC), _fixed),
            pl.BlockSpec((1, ENC), _fixed),
            pl.BlockSpec((ENC, N_INTENTS), _fixed),
            pl.BlockSpec((1, N_INTENTS), _fixed),
            pl.BlockSpec((slot_dec_W.shape[0], N_SLOTS), _fixed),
            pl.BlockSpec((1, N_SLOTS), _fixed),
        ],
        out_specs=[
            pl.BlockSpec((C, 1, N_INTENTS), _intent_map),
            pl.BlockSpec((C * S, N_SLOTS), _slots_map),
        ],
        scratch_shapes=[
            pltpu.VMEM((half * S, EMB), jnp.float32),
            pltpu.SemaphoreType.DMA((nj,)),
        ],
    )

    intent_batch, slots_batch = pl.pallas_call(
        _body,
        out_shape=[
            jax.ShapeDtypeStruct((B, 1, N_INTENTS), jnp.float32),
            jax.ShapeDtypeStruct((B * S, N_SLOTS), jnp.float32),
        ],
        grid_spec=grid_spec,
        compiler_params=pltpu.CompilerParams(
            dimension_semantics=("parallel", "arbitrary"),
            disable_bounds_checks=True,
        ),
        name="intent_slot_fused",
    )(tok, intents, emb_table, doc_enc_W, doc_enc_b.reshape(1, ENC),
      slot_enc_W, slot_enc_b.reshape(1, ENC), doc_dec_W,
      doc_dec_b.reshape(1, N_INTENTS), slot_dec_W,
      slot_dec_b.reshape(1, N_SLOTS))
    return intent_batch.reshape(B, N_INTENTS), slots_batch


# speedup vs baseline: 1.0681x; 1.0169x over previous
import jax
import jax.numpy as jnp
from jax import lax
from jax.experimental import pallas as pl
from jax.experimental.pallas import tpu as pltpu

_NCH = 4


def _body(tok_ref, intent_ref, emb_hbm,
          dW_ref, db_ref, sW_ref, sb_ref, ddW_ref, ddb_ref, sdW_ref, sdb_ref,
          intent_out, slots_out, xbuf, sems):
    nj = pl.num_programs(1)
    S = tok_ref.shape[1]
    rows = xbuf.shape[0]
    CS = rows // nj
    C = CS // S
    c = pl.program_id(0)
    j = pl.program_id(1)
    b0 = c * (nj * C)

    @pl.when(j == 0)
    def _issue_all():
        for wi in range(rows):
            tok = tok_ref[b0 + wi // S, wi % S]
            pltpu.make_async_copy(
                emb_hbm.at[pl.ds(tok, 1), :],
                xbuf.at[pl.ds(wi, 1), :],
                sems.at[wi // CS]).start()

    pltpu.make_async_copy(emb_hbm.at[pl.ds(0, CS), :],
                          xbuf.at[pl.ds(0, CS), :], sems.at[j]).wait()

    base = pl.multiple_of(j * CS, 8)
    x = xbuf[pl.ds(base, CS), :]

    we = jnp.maximum(
        jnp.dot(x, sW_ref[...], preferred_element_type=jnp.float32)
        + sb_ref[...], 0.0)

    top = jnp.dot(we, sdW_ref[0:256, :], preferred_element_type=jnp.float32)

    first = b0 + j * C
    n_int = sdW_ref.shape[0] - 256
    iota = lax.broadcasted_iota(jnp.int32, (1, n_int), 1)
    oh = jnp.concatenate(
        [(iota == intent_ref[first + r]).astype(jnp.float32)
         for r in range(C)], axis=0)
    bots = jnp.dot(oh, sdW_ref[256:, :], preferred_element_type=jnp.float32)

    n_slots = top.shape[1]
    top3 = top.reshape(C, S, n_slots)
    slots_out[...] = (top3 + bots[:, None, :] + sdb_ref[...]).reshape(CS, n_slots)

    m = jnp.mean(x.reshape(C, S, x.shape[1]), axis=1)
    se = jnp.maximum(
        jnp.dot(m, dW_ref[...], preferred_element_type=jnp.float32)
        + db_ref[...], 0.0)
    logits = (jnp.dot(se, ddW_ref[...], preferred_element_type=jnp.float32)
              + ddb_ref[...])
    intent_out[...] = logits.reshape(intent_out.shape)


def kernel(token_ids, all_intents, emb_table, doc_enc_W, doc_enc_b,
           slot_enc_W, slot_enc_b, doc_dec_W, doc_dec_b,
           slot_dec_W, slot_dec_b):
    B, S = token_ids.shape
    VOCAB, EMB = emb_table.shape
    ENC = doc_enc_W.shape[1]
    N_INTENTS = doc_dec_W.shape[1]
    N_SLOTS = slot_dec_W.shape[1]
    nj = _NCH
    half = B // 2
    C = half // nj

    tok = token_ids.astype(jnp.int32)
    intents = all_intents.astype(jnp.int32)

    grid = (2, nj)

    def _fixed(c, j, *_):
        return (0, 0)

    def _slots_map(c, j, *_):
        return (c * nj + j, 0)

    def _intent_map(c, j, *_):
        return (c * nj + j, 0, 0)

    grid_spec = pltpu.PrefetchScalarGridSpec(
        num_scalar_prefetch=2,
        grid=grid,
        in_specs=[
            pl.BlockSpec(memory_space=pl.ANY),
            pl.BlockSpec((EMB, ENC), _fixed),
            pl.BlockSpec((1, ENC), _fixed),
            pl.BlockSpec((EMB, ENC), _fixed),
            pl.BlockSpec((1, ENC), _fixed),
            pl.BlockSpec((ENC, N_INTENTS), _fixed),
            pl.BlockSpec((1, N_INTENTS), _fixed),
            pl.BlockSpec((slot_dec_W.shape[0], N_SLOTS), _fixed),
            pl.BlockSpec((1, N_SLOTS), _fixed),
        ],
        out_specs=[
            pl.BlockSpec((C, 1, N_INTENTS), _intent_map),
            pl.BlockSpec((C * S, N_SLOTS), _slots_map),
        ],
        scratch_shapes=[
            pltpu.VMEM((half * S, EMB), jnp.float32),
            pltpu.SemaphoreType.DMA((nj,)),
        ],
    )

    intent_batch, slots_batch = pl.pallas_call(
        _body,
        out_shape=[
            jax.ShapeDtypeStruct((B, 1, N_INTENTS), jnp.float32),
            jax.ShapeDtypeStruct((B * S, N_SLOTS), jnp.float32),
        ],
        grid_spec=grid_spec,
        compiler_params=pltpu.CompilerParams(
            dimension_semantics=("parallel", "arbitrary"),
            disable_bounds_checks=True,
        ),
        name="intent_slot_fused",
    )(tok, intents, emb_table, doc_enc_W, doc_enc_b.reshape(1, ENC),
      slot_enc_W, slot_enc_b.reshape(1, ENC), doc_dec_W,
      doc_dec_b.reshape(1, N_INTENTS), slot_dec_W,
      slot_dec_b.reshape(1, N_SLOTS))
    return intent_batch.reshape(B, N_INTENTS), slots_batch
